# baseline (speedup 1.0000x reference)
import functools

import jax
import jax.numpy as jnp
from jax import lax
from jax.experimental import pallas as pl
from jax.experimental.pallas import tpu as pltpu
from jax.experimental.pallas import tpu_sc as plsc

EMBED_DIM = 128
BLK = 128
NBUF = 5


@functools.cache
def _make_gather(total: int):
    info = plsc.get_sparse_core_info()
    nc, ns = info.num_cores, info.num_subcores
    nw = nc * ns
    assert total % (nw * BLK * NBUF) == 0
    per_w = total // nw
    n_b = per_w // BLK
    n_g = n_b // NBUF
    assert n_g >= 2

    mesh = plsc.VectorSubcoreMesh(core_axis_name="c", subcore_axis_name="s")

    @functools.partial(
        pl.kernel,
        out_type=jax.ShapeDtypeStruct((total, EMBED_DIM), jnp.float32),
        mesh=mesh,
        scratch_types=(
            [pltpu.VMEM((per_w,), jnp.int32)]
            + [pltpu.VMEM((BLK, EMBED_DIM), jnp.float32)] * NBUF
            + [pltpu.SemaphoreType.DMA] * (2 * NBUF)
        ),
    )
    def gather(table_hbm, idx_hbm, out_hbm, idx_v, *rest):
        bufs = rest[:NBUF]
        gsems = rest[NBUF:2 * NBUF]
        wsems = rest[2 * NBUF:]
        wid = lax.axis_index("s") * nc + lax.axis_index("c")
        base = wid * per_w
        pltpu.sync_copy(idx_hbm.at[pl.ds(base, per_w)], idx_v)

        def g_start(i, b):
            pltpu.async_copy(
                table_hbm.at[idx_v.at[pl.ds(i * BLK, BLK)]], bufs[b], gsems[b]
            )

        def w_start(i, b):
            pltpu.async_copy(
                bufs[b], out_hbm.at[pl.ds(base + i * BLK, BLK)], wsems[b]
            )

        def drain(sem, b):
            pltpu.make_async_copy(table_hbm.at[pl.ds(0, BLK)], bufs[b], sem).wait()

        for b in range(NBUF):
            g_start(b, b)

        def body(g, carry):
            i0 = g * NBUF
            for b in range(NBUF):
                drain(gsems[b], b)
                w_start(i0 + b, b)
            for b in range(NBUF):
                drain(wsems[b], b)
                g_start(i0 + NBUF + b, b)
            return carry

        lax.fori_loop(0, n_g - 1, body, 0)

        i0 = (n_g - 1) * NBUF
        for b in range(NBUF):
            drain(gsems[b], b)
            w_start(i0 + b, b)
        for b in range(NBUF):
            drain(wsems[b], b)

    return gather


def kernel(triples, norm_vector_weight):
    b, n, k = triples.shape
    assert k == 5
    idx = triples[:, :, 3].reshape(-1).astype(jnp.int32)
    out = _make_gather(b * n)(norm_vector_weight, idx)
    return out.reshape(b, n, 1, 1, EMBED_DIM)


# speedup vs baseline: 1.2338x; 1.2338x over previous
import functools

import jax
import jax.numpy as jnp
from jax import lax
from jax.experimental import pallas as pl
from jax.experimental.pallas import tpu as pltpu
from jax.experimental.pallas import tpu_sc as plsc

EMBED_DIM = 128
BLK = 128
NBUF = 2


@functools.cache
def _make_gather(total: int, rows_pad: int):
    info = plsc.get_sparse_core_info()
    nc, ns = info.num_cores, info.num_subcores
    nw = nc * ns
    assert total % (nw * BLK * NBUF) == 0
    assert rows_pad % ns == 0
    stage_rows = rows_pad // ns
    per_w = total // nw
    n_b = per_w // BLK
    n_g = n_b // NBUF
    assert n_g >= 2

    mesh = plsc.VectorSubcoreMesh(core_axis_name="c", subcore_axis_name="s")

    @functools.partial(
        pl.kernel,
        out_type=jax.ShapeDtypeStruct((total, EMBED_DIM), jnp.float32),
        mesh=mesh,
        scratch_types=(
            [pltpu.VMEM_SHARED((rows_pad, EMBED_DIM), jnp.float32)]
            + [pltpu.VMEM((stage_rows, EMBED_DIM), jnp.float32)]
            + [pltpu.VMEM((per_w,), jnp.int32)]
            + [pltpu.VMEM((BLK, EMBED_DIM), jnp.float32)] * NBUF
            + [pltpu.SemaphoreType.DMA] * (2 * NBUF)
        ),
    )
    def gather(table_hbm, idx_hbm, out_hbm, table_sp, stage_v, idx_v, *rest):
        bufs = rest[:NBUF]
        gsems = rest[NBUF:2 * NBUF]
        wsems = rest[2 * NBUF:]
        sid = lax.axis_index("s")
        wid = sid * nc + lax.axis_index("c")
        base = wid * per_w

        row0 = sid * stage_rows
        pltpu.sync_copy(table_hbm.at[pl.ds(row0, stage_rows)], stage_v)
        pltpu.sync_copy(stage_v, table_sp.at[pl.ds(row0, stage_rows)])
        pltpu.sync_copy(idx_hbm.at[pl.ds(base, per_w)], idx_v)
        plsc.subcore_barrier()

        def g_start(i, b):
            pltpu.async_copy(
                table_sp.at[idx_v.at[pl.ds(i * BLK, BLK)]], bufs[b], gsems[b]
            )

        def w_start(i, b):
            pltpu.async_copy(
                bufs[b], out_hbm.at[pl.ds(base + i * BLK, BLK)], wsems[b]
            )

        def drain(sem, b):
            pltpu.make_async_copy(table_hbm.at[pl.ds(0, BLK)], bufs[b], sem).wait()

        for b in range(NBUF):
            g_start(b, b)

        def body(g, carry):
            i0 = g * NBUF
            for b in range(NBUF):
                drain(gsems[b], b)
                w_start(i0 + b, b)
            for b in range(NBUF):
                drain(wsems[b], b)
                g_start(i0 + NBUF + b, b)
            return carry

        lax.fori_loop(0, n_g - 1, body, 0)

        i0 = (n_g - 1) * NBUF
        for b in range(NBUF):
            drain(gsems[b], b)
            w_start(i0 + b, b)
        for b in range(NBUF):
            drain(wsems[b], b)

    return gather


def kernel(triples, norm_vector_weight):
    b, n, k = triples.shape
    assert k == 5
    idx = triples[:, :, 3].reshape(-1).astype(jnp.int32)
    rows, d = norm_vector_weight.shape
    assert d == EMBED_DIM
    rows_pad = (rows + 127) // 128 * 128
    table = jnp.pad(norm_vector_weight, ((0, rows_pad - rows), (0, 0)))
    out = _make_gather(b * n, rows_pad)(table, idx)
    return out.reshape(b, n, 1, 1, EMBED_DIM)


# speedup vs baseline: 1.5921x; 1.2904x over previous
import functools

import jax
import jax.numpy as jnp
from jax import lax
from jax.experimental import pallas as pl
from jax.experimental.pallas import tpu as pltpu
from jax.experimental.pallas import tpu_sc as plsc

EMBED_DIM = 128
BLK = 128
GPB = 2
NBUF = 2


@functools.cache
def _make_gather(total: int, rows_pad: int):
    info = plsc.get_sparse_core_info()
    nc, ns = info.num_cores, info.num_subcores
    nw = nc * ns
    rows_buf = GPB * BLK
    assert total % (nw * rows_buf) == 0
    assert rows_pad % ns == 0
    stage_rows = rows_pad // ns
    assert stage_rows <= rows_buf
    per_w = total // nw
    n_b = per_w // rows_buf
    assert n_b >= NBUF

    mesh = plsc.VectorSubcoreMesh(core_axis_name="c", subcore_axis_name="s")

    @functools.partial(
        pl.kernel,
        out_type=jax.ShapeDtypeStruct((total, EMBED_DIM), jnp.float32),
        mesh=mesh,
        scratch_types=(
            [pltpu.VMEM_SHARED((rows_pad, EMBED_DIM), jnp.float32)]
            + [pltpu.VMEM((per_w,), jnp.int32)]
            + [pltpu.VMEM((rows_buf, EMBED_DIM), jnp.float32)] * NBUF
            + [pltpu.SemaphoreType.DMA] * (2 * NBUF)
        ),
    )
    def gather(table_hbm, idx_hbm, out_hbm, table_sp, idx_v, *rest):
        bufs = rest[:NBUF]
        gsems = rest[NBUF:2 * NBUF]
        wsems = rest[2 * NBUF:]
        sid = lax.axis_index("s")
        wid = sid * nc + lax.axis_index("c")
        base = wid * per_w

        row0 = sid * stage_rows
        pltpu.sync_copy(table_hbm.at[pl.ds(row0, stage_rows)],
                        bufs[0].at[pl.ds(0, stage_rows)])
        pltpu.sync_copy(bufs[0].at[pl.ds(0, stage_rows)],
                        table_sp.at[pl.ds(row0, stage_rows)])
        pltpu.sync_copy(idx_hbm.at[pl.ds(base, per_w)], idx_v)
        plsc.subcore_barrier()

        def g_start(i, b):
            for g in range(GPB):
                pltpu.async_copy(
                    table_sp.at[idx_v.at[pl.ds((i * GPB + g) * BLK, BLK)]],
                    bufs[b].at[pl.ds(g * BLK, BLK)],
                    gsems[b],
                )

        def w_start(i, b):
            pltpu.async_copy(
                bufs[b], out_hbm.at[pl.ds(base + i * rows_buf, rows_buf)],
                wsems[b],
            )

        def g_drain(b):
            pltpu.make_async_copy(
                table_hbm.at[pl.ds(0, rows_buf)], bufs[b], gsems[b]
            ).wait()

        def w_drain(b):
            pltpu.make_async_copy(
                table_hbm.at[pl.ds(0, rows_buf)], bufs[b], wsems[b]
            ).wait()

        for i in range(n_b):
            b = i % NBUF
            if i >= NBUF:
                w_drain(b)
            g_start(i, b)
            g_drain(b)
            w_start(i, b)
        for i in range(NBUF):
            w_drain((n_b - NBUF + i) % NBUF)

    return gather


def kernel(triples, norm_vector_weight):
    b, n, k = triples.shape
    assert k == 5
    idx = triples[:, :, 3].reshape(-1).astype(jnp.int32)
    rows, d = norm_vector_weight.shape
    assert d == EMBED_DIM
    rows_pad = (rows + 127) // 128 * 128
    table = jnp.pad(norm_vector_weight, ((0, rows_pad - rows), (0, 0)))
    out = _make_gather(b * n, rows_pad)(table, idx)
    return out.reshape(b, n, 1, 1, EMBED_DIM)


# speedup vs baseline: 1.6147x; 1.0142x over previous
import functools

import jax
import jax.numpy as jnp
from jax import lax
from jax.experimental import pallas as pl
from jax.experimental.pallas import tpu as pltpu
from jax.experimental.pallas import tpu_sc as plsc

EMBED_DIM = 128
BLK = 128
GPB = 2
NBUF = 2


@functools.cache
def _make_gather(total: int, rows_pad: int):
    info = plsc.get_sparse_core_info()
    nc, ns = info.num_cores, info.num_subcores
    nw = nc * ns
    rows_buf = GPB * BLK
    assert total % (nw * rows_buf) == 0
    assert rows_pad % ns == 0
    stage_rows = rows_pad // ns
    assert stage_rows <= rows_buf
    per_w = total // nw
    n_b = per_w // rows_buf
    assert n_b >= NBUF

    mesh = plsc.VectorSubcoreMesh(core_axis_name="c", subcore_axis_name="s")

    @functools.partial(
        pl.kernel,
        out_type=jax.ShapeDtypeStruct((total, EMBED_DIM), jnp.float32),
        mesh=mesh,
        scratch_types=(
            [pltpu.VMEM_SHARED((rows_pad, EMBED_DIM), jnp.float32)]
            + [pltpu.VMEM((per_w,), jnp.int32)]
            + [pltpu.VMEM((rows_buf, EMBED_DIM), jnp.float32)] * NBUF
            + [pltpu.SemaphoreType.DMA] * (2 * NBUF)
        ),
    )
    def gather(table_hbm, idx_hbm, out_hbm, table_sp, idx_v, *rest):
        bufs = rest[:NBUF]
        gsems = rest[NBUF:2 * NBUF]
        wsems = rest[2 * NBUF:]
        sid = lax.axis_index("s")
        wid = sid * nc + lax.axis_index("c")
        base = wid * per_w

        row0 = sid * stage_rows
        half = stage_rows // 2
        pltpu.async_copy(idx_hbm.at[pl.ds(base, per_w)], idx_v, wsems[0])
        for h in range(2):
            pltpu.async_copy(
                table_hbm.at[pl.ds(row0 + h * half, half)],
                bufs[h].at[pl.ds(0, half)],
                gsems[h],
            )
        for h in range(2):
            pltpu.make_async_copy(
                table_hbm.at[pl.ds(0, half)], bufs[h].at[pl.ds(0, half)],
                gsems[h],
            ).wait()
            pltpu.async_copy(
                bufs[h].at[pl.ds(0, half)],
                table_sp.at[pl.ds(row0 + h * half, half)],
                gsems[h],
            )
        for h in range(2):
            pltpu.make_async_copy(
                table_hbm.at[pl.ds(0, half)], bufs[h].at[pl.ds(0, half)],
                gsems[h],
            ).wait()
        plsc.subcore_barrier()
        pltpu.make_async_copy(
            idx_hbm.at[pl.ds(0, per_w)], idx_v, wsems[0]
        ).wait()

        def g_start(i, b):
            for g in range(GPB):
                pltpu.async_copy(
                    table_sp.at[idx_v.at[pl.ds((i * GPB + g) * BLK, BLK)]],
                    bufs[b].at[pl.ds(g * BLK, BLK)],
                    gsems[b],
                )

        def w_start(i, b):
            pltpu.async_copy(
                bufs[b], out_hbm.at[pl.ds(base + i * rows_buf, rows_buf)],
                wsems[b],
            )

        def g_drain(b):
            pltpu.make_async_copy(
                table_hbm.at[pl.ds(0, rows_buf)], bufs[b], gsems[b]
            ).wait()

        def w_drain(b):
            pltpu.make_async_copy(
                table_hbm.at[pl.ds(0, rows_buf)], bufs[b], wsems[b]
            ).wait()

        for i in range(n_b):
            b = i % NBUF
            if i >= NBUF:
                w_drain(b)
            g_start(i, b)
            g_drain(b)
            w_start(i, b)
        for i in range(NBUF):
            w_drain((n_b - NBUF + i) % NBUF)

    return gather


def kernel(triples, norm_vector_weight):
    b, n, k = triples.shape
    assert k == 5
    idx = triples[:, :, 3].reshape(-1).astype(jnp.int32)
    rows, d = norm_vector_weight.shape
    assert d == EMBED_DIM
    rows_pad = (rows + 127) // 128 * 128
    table = jnp.pad(norm_vector_weight, ((0, rows_pad - rows), (0, 0)))
    out = _make_gather(b * n, rows_pad)(table, idx)
    return out.reshape(b, n, 1, 1, EMBED_DIM)
